# baseline (speedup 1.0000x reference)
import functools

import jax
import jax.numpy as jnp
import numpy as np
from jax import lax
from jax.experimental import pallas as pl
from jax.experimental.pallas import tpu as pltpu

B = 32
V = 1_000_000
VPAD = 1 << 20
C = 16384
NC = VPAD // C
CUTOFF = np.float32(0.9)

_I32 = jnp.int32
_TINY = np.float32(np.finfo(np.float32).tiny)


def _rotl(x, r):
  return lax.shift_left(x, np.int32(r)) | lax.shift_right_logical(
      x, np.int32(32 - r))


def _threefry_bits(x1):
  ks0 = np.int32(0)
  ks1 = np.int32(42)
  ks2 = np.int32(np.uint32(0 ^ 42 ^ 0x1BD11BDA).astype(np.int32))
  r0 = (13, 15, 26, 6)
  r1 = (17, 29, 16, 24)

  x0 = jnp.full_like(x1, ks0)
  x1 = x1 + ks1

  def rounds(x0, x1, rots):
    for r in rots:
      x0 = x0 + x1
      x1 = _rotl(x1, r)
      x1 = x0 ^ x1
    return x0, x1

  x0, x1 = rounds(x0, x1, r0)
  x0, x1 = x0 + ks1, x1 + ks2 + np.int32(1)
  x0, x1 = rounds(x0, x1, r1)
  x0, x1 = x0 + ks2, x1 + ks0 + np.int32(2)
  x0, x1 = rounds(x0, x1, r0)
  x0, x1 = x0 + ks0, x1 + ks1 + np.int32(3)
  x0, x1 = rounds(x0, x1, r1)
  x0, x1 = x0 + ks1, x1 + ks2 + np.int32(4)
  x0, x1 = rounds(x0, x1, r0)
  x0, x1 = x0 + ks2, x1 + ks0 + np.int32(5)
  return x0 ^ x1


def _gumbel(flat_idx_i32):
  bits = _threefry_bits(flat_idx_i32)
  fb = lax.shift_right_logical(bits, np.int32(9)) | np.int32(0x3F800000)
  f = lax.bitcast_convert_type(fb, jnp.float32) - np.float32(1.0)
  u = jnp.maximum(_TINY, f + _TINY)
  return -jnp.log(-jnp.log(u))


def _sample_body(sv_ref, out_ref, state):
  r = pl.program_id(0)
  p = pl.program_id(1)
  c = pl.program_id(2)

  rows = C // 128
  jj = (c * C + jax.lax.broadcasted_iota(_I32, (rows, 128), 0) * 128
        + jax.lax.broadcasted_iota(_I32, (rows, 128), 1))
  valid = jj < V
  v = jnp.where(valid, sv_ref[0], np.float32(0.0))

  x = v
  ri = jax.lax.broadcasted_iota(_I32, (rows, rows), 0)
  ci = jax.lax.broadcasted_iota(_I32, (rows, rows), 1)
  lstrict = (ri > ci).astype(jnp.float32)
  li = jax.lax.broadcasted_iota(_I32, (128, 128), 0)
  lj = jax.lax.broadcasted_iota(_I32, (128, 128), 1)
  ut = (li <= lj).astype(jnp.float32)
  lanecum = jax.lax.dot_general(
      x, ut, (((1,), (0,)), ((), ())), preferred_element_type=jnp.float32)
  sub = jax.lax.dot_general(
      lstrict, x, (((1,), (0,)), ((), ())), preferred_element_type=jnp.float32)
  cum_in = lanecum + jnp.sum(sub, axis=1, keepdims=True)

  @pl.when((p == 0) & (c == 0))
  def _():
    state[0] = np.float32(0.0)
    state[1] = np.float32(0.0)

  @pl.when(p == 0)
  def _():
    carry = state[0]
    cum = cum_in + carry
    kept = cum <= CUTOFF
    state[1] = state[1] + jnp.sum(jnp.where(kept, v, np.float32(0.0)))
    state[0] = carry + jnp.sum(v)

  @pl.when((p == 1) & (c == 0))
  def _():
    state[2] = np.float32(-np.inf)
    state[3] = np.float32(2.0e9)
    state[0] = np.float32(0.0)

  @pl.when(p == 1)
  def _():
    carry = state[0]
    total = jnp.maximum(state[1], np.float32(1e-10))
    cum = cum_in + carry
    kept = cum <= CUTOFF
    w = jnp.where(kept, v / total, np.float32(0.0))
    flat = r * np.int32(V) + jj
    g = _gumbel(flat)
    s = jnp.log(w + np.float32(1e-10)) + g
    s = jnp.where(valid, s, np.float32(-np.inf))
    m = jnp.max(s)
    jl = jnp.min(jnp.where(s == m, jj, np.int32(2**31 - 1))).astype(jnp.float32)
    best = state[2]
    bestj = state[3]
    better = (m > best) | ((m == best) & (jl < bestj))
    state[2] = jnp.where(better, m, best)
    state[3] = jnp.where(better, jl, bestj)
    state[0] = carry + jnp.sum(v)

  @pl.when((p == 1) & (c == NC - 1))
  def _():
    out_ref[0, r] = state[3].astype(_I32)


def _sample_rank(svals):
  sv3 = svals.reshape(B * NC, C // 128, 128)
  out = pl.pallas_call(
      _sample_body,
      grid=(B, 2, NC),
      in_specs=[
          pl.BlockSpec((1, C // 128, 128), lambda r, p, c: (r * NC + c, 0, 0))
      ],
      out_specs=pl.BlockSpec(memory_space=pltpu.SMEM),
      out_shape=jax.ShapeDtypeStruct((1, B), _I32),
      scratch_shapes=[pltpu.SMEM((8,), jnp.float32)],
  )(sv3)
  return out[0]


def _sort_descending(probs):
  order = jnp.argsort(-probs, axis=-1)
  sv = jnp.take_along_axis(probs, order, axis=-1)
  pad = ((0, 0), (0, VPAD - V))
  return (jnp.pad(sv, pad), jnp.pad(order.astype(_I32), pad))


def kernel(logits):
  probs = jax.nn.softmax(logits, axis=-1)
  svals, sidx = _sort_descending(probs)
  jstar = _sample_rank(svals)
  tok = jnp.take_along_axis(sidx, jstar[:, None], axis=-1)[:, 0]
  return tok
